# Initial kernel scaffold; baseline (speedup 1.0000x reference)
import jax
import jax.numpy as jnp
from jax.experimental import pallas as pl


def kernel(x_inp, edge_index, edge_attr, W_node, b_node, W_edge, b_edge, W_msg0, b_msg0, W_upd0, b_upd0, W_msg1, b_msg1, W_upd1, b_upd1, W_fin, b_fin):
    raise NotImplementedError("write your pallas kernel here")


# speedup vs baseline: 3.0120x; 3.0120x over previous
import functools

import jax
import jax.numpy as jnp
from jax import lax
from jax.experimental import pallas as pl
from jax.experimental.pallas import tpu as pltpu
from jax.experimental.pallas import tpu_sc as plsc

N = 10000
E = 320000
D_NODE = 128
D_EDGE = 16
H = 64

NC = 2
NS = 16
NW = NC * NS
EPW = E // NW
CH = 80
NCH = EPW // CH
N_PAD = 10240
RPS = N_PAD // NS
ZR = 32
LANES = 16

RN = 1000
RE = 4000


def _silu(v):
    return v * jax.nn.sigmoid(v)


def _dot(a, b):
    return jnp.dot(a, b, preferred_element_type=jnp.float32)


def _embed_proj_body(x_ref, wn_ref, bn_ref, wsd_ref, bm_ref,
                     x0_ref, ab_ref):
    x0 = _silu(_dot(x_ref[...], wn_ref[...]) + bn_ref[...])
    x0_ref[...] = x0
    ab_ref[...] = _dot(x0, wsd_ref[...]) + bm_ref[...]


def _edge_body(ea_ref, we_ref, be_ref, w01_ref, c01_ref):
    e = _silu(_dot(ea_ref[...], we_ref[...]) + be_ref[...])
    c01_ref[...] = _dot(e, w01_ref[...])


def _update_proj_body(x_ref, agg_ref, wux_ref, wua_ref, bu_ref,
                      wsd_ref, bm_ref, x1_ref, ab_ref):
    agg = (agg_ref[0] + agg_ref[1])[:, :H]
    x = x_ref[...]
    x1 = x + _silu(_dot(x, wux_ref[...]) + _dot(agg, wua_ref[...]) + bu_ref[...])
    x1_ref[...] = x1
    ab_ref[...] = _dot(x1, wsd_ref[...]) + bm_ref[...]


def _final_body(x_ref, agg_ref, wux_ref, wua_ref, bu_ref,
                xin_ref, wfx_ref, wfi_ref, bf_ref, out_ref):
    agg = (agg_ref[0] + agg_ref[1])[:, :H]
    x = x_ref[...]
    x2 = x + _silu(_dot(x, wux_ref[...]) + _dot(agg, wua_ref[...]) + bu_ref[...])
    out_ref[...] = _silu(_dot(x2, wfx_ref[...]) + _dot(xin_ref[...], wfi_ref[...])
                         + bf_ref[...])


def _row_spec(rows, cols):
    return pl.BlockSpec((rows, cols), lambda i: (i, 0))


def _full_spec(shape):
    return pl.BlockSpec(shape, lambda i: tuple(0 for _ in shape))


_fmt = jnp.float32


def _embed_proj(x_inp, wn, bn, wsd, bm):
    return pl.pallas_call(
        _embed_proj_body,
        grid=(N // RN,),
        in_specs=[_row_spec(RN, D_NODE), _full_spec((D_NODE, H)),
                  _full_spec((1, H)), _full_spec((H, 2 * H)),
                  _full_spec((1, 2 * H))],
        out_specs=[_row_spec(RN, H), _row_spec(RN, 2 * H)],
        out_shape=[jax.ShapeDtypeStruct((N, H), _fmt),
                   jax.ShapeDtypeStruct((N, 2 * H), _fmt)],
    )(x_inp, wn, bn, wsd, bm)


def _edge_proj(edge_attr, we, be, w01):
    return pl.pallas_call(
        _edge_body,
        grid=(E // RE,),
        in_specs=[_row_spec(RE, D_EDGE), _full_spec((D_EDGE, H)),
                  _full_spec((1, H)), _full_spec((H, 2 * H))],
        out_specs=[_row_spec(RE, 2 * H)],
        out_shape=[jax.ShapeDtypeStruct((E, 2 * H), _fmt)],
    )(edge_attr, we, be, w01)[0]


def _update_proj(x, agg2, wux, wua, bu, wsd, bm):
    return pl.pallas_call(
        _update_proj_body,
        grid=(N // RN,),
        in_specs=[_row_spec(RN, H),
                  pl.BlockSpec((2, RN, 2 * H), lambda i: (0, i, 0)),
                  _full_spec((H, H)), _full_spec((H, H)), _full_spec((1, H)),
                  _full_spec((H, 2 * H)), _full_spec((1, 2 * H))],
        out_specs=[_row_spec(RN, H), _row_spec(RN, 2 * H)],
        out_shape=[jax.ShapeDtypeStruct((N, H), _fmt),
                   jax.ShapeDtypeStruct((N, 2 * H), _fmt)],
    )(x, agg2, wux, wua, bu, wsd, bm)


def _final(x, agg2, wux, wua, bu, x_inp, wfx, wfi, bf):
    return pl.pallas_call(
        _final_body,
        grid=(N // RN,),
        in_specs=[_row_spec(RN, H),
                  pl.BlockSpec((2, RN, 2 * H), lambda i: (0, i, 0)),
                  _full_spec((H, H)), _full_spec((H, H)), _full_spec((1, H)),
                  _row_spec(RN, D_NODE), _full_spec((H, H)),
                  _full_spec((D_NODE, H)), _full_spec((1, H))],
        out_specs=[_row_spec(RN, H)],
        out_shape=[jax.ShapeDtypeStruct((N, H), _fmt)],
    )(x, agg2, wux, wua, bu, x_inp, wfx, wfi, bf)[0]


def _edge_pass_body(coff, ab_hbm, c_hbm, src_hbm, dst_hbm, out_hbm,
                    srcc, dstc, av, bv, cv, stage, agg_sh, s1, s2, s3):
    c = lax.axis_index("c")
    s = lax.axis_index("s")
    wid = s * NC + c
    base_e = wid * EPW

    def _zero_row(r, _):
        for q in range(2 * H // LANES):
            stage[r, pl.ds(q * LANES, LANES)] = jnp.zeros((LANES,), _fmt)
        return 0
    lax.fori_loop(0, ZR, _zero_row, 0)

    def _zero_slab(t, _):
        pltpu.sync_copy(stage, agg_sh.at[pl.ds(s * RPS + t * ZR, ZR)])
        return 0
    lax.fori_loop(0, RPS // ZR, _zero_slab, 0)
    plsc.subcore_barrier()

    def _chunk(j, _):
        eoff = base_e + j * CH
        pltpu.sync_copy(src_hbm.at[pl.ds(eoff, CH)], srcc)
        pltpu.sync_copy(dst_hbm.at[pl.ds(eoff, CH)], dstc)
        ga = pltpu.async_copy(ab_hbm.at[srcc], av, s1)
        gb = pltpu.async_copy(ab_hbm.at[dstc], bv, s2)
        gc = pltpu.async_copy(c_hbm.at[pl.ds(eoff, CH)], cv, s3)
        ga.wait()
        gb.wait()
        gc.wait()

        def _row(r, _):
            for q in range(H // LANES):
                sl = pl.ds(q * LANES, LANES)
                v = (av[r, sl] + bv[r, pl.ds(H + q * LANES, LANES)]
                     + cv[r, pl.ds(coff + q * LANES, LANES)])
                cv[r, sl] = v / (1.0 + jnp.exp(-v))
            return 0
        lax.fori_loop(0, CH, _row, 0)

        pltpu.sync_copy(cv, agg_sh.at[dstc], add=True)
        return 0

    lax.fori_loop(0, NCH, _chunk, 0)
    plsc.subcore_barrier()

    pltpu.sync_copy(agg_sh.at[pl.ds(s * RPS, RPS)], out_hbm.at[c, s])


def _make_edge_pass(coff):
    @functools.partial(
        pl.kernel,
        out_type=jax.ShapeDtypeStruct((NC, NS, RPS, 2 * H), jnp.float32),
        mesh=plsc.VectorSubcoreMesh(core_axis_name="c", subcore_axis_name="s"),
        scratch_types=[
            pltpu.VMEM((CH,), jnp.int32),
            pltpu.VMEM((CH,), jnp.int32),
            pltpu.VMEM((CH, 2 * H), jnp.float32),
            pltpu.VMEM((CH, 2 * H), jnp.float32),
            pltpu.VMEM((CH, 2 * H), jnp.float32),
            pltpu.VMEM((ZR, 2 * H), jnp.float32),
            pltpu.VMEM_SHARED((N_PAD, 2 * H), jnp.float32),
            pltpu.SemaphoreType.DMA,
            pltpu.SemaphoreType.DMA,
            pltpu.SemaphoreType.DMA,
        ],
    )
    def _edge_pass(ab_hbm, c_hbm, src_hbm, dst_hbm, out_hbm,
                   srcc, dstc, av, bv, cv, stage, agg_sh, s1, s2, s3):
        _edge_pass_body(coff, ab_hbm, c_hbm, src_hbm, dst_hbm, out_hbm,
                        srcc, dstc, av, bv, cv, stage, agg_sh, s1, s2, s3)
    return _edge_pass


_edge_pass0 = _make_edge_pass(0)
_edge_pass1 = _make_edge_pass(H)


def kernel(x_inp, edge_index, edge_attr, W_node, b_node, W_edge, b_edge,
           W_msg0, b_msg0, W_upd0, b_upd0, W_msg1, b_msg1, W_upd1, b_upd1,
           W_fin, b_fin):
    f32 = jnp.float32
    src1d = edge_index[0].astype(jnp.int32)
    dst1d = edge_index[1].astype(jnp.int32)

    bn = b_node.reshape(1, H)
    be = b_edge.reshape(1, H)
    bm0 = b_msg0.reshape(1, H)
    bm1 = b_msg1.reshape(1, H)
    bu0 = b_upd0.reshape(1, H)
    bu1 = b_upd1.reshape(1, H)
    bf = b_fin.reshape(1, H)

    w01e = jnp.concatenate([W_msg0[2 * H:], W_msg1[2 * H:]], axis=1)
    wsd0 = jnp.concatenate([W_msg0[:H], W_msg0[H:2 * H]], axis=1)
    wsd1 = jnp.concatenate([W_msg1[:H], W_msg1[H:2 * H]], axis=1)
    bsd0 = jnp.concatenate([jnp.zeros((1, H), f32), bm0], axis=1)
    bsd1 = jnp.concatenate([jnp.zeros((1, H), f32), bm1], axis=1)
    wu0x, wu0a = W_upd0[:H], W_upd0[H:]
    wu1x, wu1a = W_upd1[:H], W_upd1[H:]
    wfx, wfi = W_fin[:H], W_fin[H:]

    x0, ab0 = _embed_proj(x_inp.astype(f32), W_node, bn, wsd0, bsd0)
    c01 = _edge_proj(edge_attr.astype(f32), W_edge, be, w01e)

    agg0 = _edge_pass0(ab0, c01, src1d, dst1d).reshape(NC, N_PAD, 2 * H)
    x1, ab1 = _update_proj(x0, agg0, wu0x, wu0a, bu0, wsd1, bsd1)
    agg1 = _edge_pass1(ab1, c01, src1d, dst1d).reshape(NC, N_PAD, 2 * H)
    return _final(x1, agg1, wu1x, wu1a, bu1, x_inp.astype(f32), wfx, wfi, bf)
